# baseline (speedup 1.0000x reference)
import functools

import jax
import jax.numpy as jnp
from jax import lax
from jax.experimental import pallas as pl
from jax.experimental.pallas import tpu as pltpu
from jax.experimental.pallas import tpu_sc as plsc

_N = 65536
_NC = 2
_NS = 16
_NW = _NC * _NS
_C = 4096
_G = _C // 16
_NSL = _N // _NS
_NBUF = 4
_AHEAD = 2


@functools.lru_cache(maxsize=None)
def _spmv(nnz):
    full = nnz // _C
    tail = nnz - full * _C
    nchunks = full + (1 if tail else 0)
    steps = -(-nchunks // _NW)
    steps = -(-steps // _NBUF) * _NBUF

    mesh = plsc.VectorSubcoreMesh(core_axis_name="c", subcore_axis_name="s")

    @functools.partial(
        pl.kernel,
        out_type=jax.ShapeDtypeStruct((_NC, _N), jnp.float32),
        mesh=mesh,
        compiler_params=pltpu.CompilerParams(needs_layout_passes=False),
        scratch_types=[
            pltpu.VMEM((_N,), jnp.float32),
            *[pltpu.VMEM((_C,), jnp.int32) for _ in range(_NBUF)],
            *[pltpu.VMEM((_C,), jnp.float32) for _ in range(_NBUF)],
            *[pltpu.VMEM((_C,), jnp.float32) for _ in range(_NBUF)],
            pltpu.VMEM_SHARED((_N,), jnp.float32),
            *[pltpu.SemaphoreType.DMA for _ in range(2 * _NBUF)],
        ],
    )
    def k(rows_hbm, cols_hbm, vals_hbm, rows_t, cols_t, vals_t,
          x_hbm, out_hbm,
          x_l, rb0, rb1, rb2, rb3, cb0, cb1, cb2, cb3,
          vb0, vb1, vb2, vb3,
          y_sh, is0, is1, is2, is3, ss0, ss1, ss2, ss3):
        rows_b = (rb0, rb1, rb2, rb3)
        cols_b = (cb0, cb1, cb2, cb3)
        vals_b = (vb0, vb1, vb2, vb3)
        prod_b = cols_b
        in_sems = (is0, is1, is2, is3)
        sc_sems = (ss0, ss1, ss2, ss3)
        c = lax.axis_index("c")
        s = lax.axis_index("s")
        w = c * _NS + s

        pltpu.sync_copy(x_hbm, x_l)
        z16 = jnp.zeros((16,), jnp.float32)

        @plsc.parallel_loop(0, _G, 1, unroll=8)
        def _z(g):
            cb0[pl.ds(g * 16, 16)] = z16

        for kk in range(_NSL // _C):
            pltpu.sync_copy(cb0, y_sh.at[pl.ds(s * _NSL + kk * _C, _C)])
        plsc.subcore_barrier()

        def fire_inputs(t, slot):
            cid = t * _NW + w
            base = cid * _C

            @pl.when(cid < full)
            def _():
                pltpu.async_copy(rows_hbm.at[pl.ds(base, _C)],
                                 rows_b[slot], in_sems[slot])
                pltpu.async_copy(cols_hbm.at[pl.ds(base, _C)],
                                 cols_b[slot], in_sems[slot])
                pltpu.async_copy(vals_hbm.at[pl.ds(base, _C)],
                                 vals_b[slot], in_sems[slot])

            if tail:
                @pl.when(cid == full)
                def _():
                    pltpu.async_copy(rows_t, rows_b[slot], in_sems[slot])
                    pltpu.async_copy(cols_t, cols_b[slot], in_sems[slot])
                    pltpu.async_copy(vals_t, vals_b[slot], in_sems[slot])

        def wait_inputs(t, slot):
            cid = t * _NW + w

            @pl.when(cid <= nchunks - 1)
            def _():
                pltpu.make_async_copy(rows_hbm.at[pl.ds(0, _C)],
                                      rows_b[slot], in_sems[slot]).wait()
                pltpu.make_async_copy(cols_hbm.at[pl.ds(0, _C)],
                                      cols_b[slot], in_sems[slot]).wait()
                pltpu.make_async_copy(vals_hbm.at[pl.ds(0, _C)],
                                      vals_b[slot], in_sems[slot]).wait()

        def compute(slot):
            @plsc.parallel_loop(0, _G, 1, unroll=8)
            def g_body(g):
                sl = pl.ds(g * 16, 16)
                idx = plsc.bitcast(cols_b[slot][sl], jnp.int32)
                xv = plsc.load_gather(x_l, [idx])
                prod_b[slot][sl] = xv * vals_b[slot][sl]

        def fire_scatter(slot):
            pltpu.async_copy(prod_b[slot], y_sh.at[rows_b[slot]],
                             sc_sems[slot], add=True)

        def wait_scatter(slot):
            pltpu.make_async_copy(prod_b[slot], y_sh.at[rows_b[slot]],
                                  sc_sems[slot]).wait()

        for t0 in range(_AHEAD):
            fire_inputs(t0, t0 % _NBUF)

        def pipe_body(i, carry):
            for j in range(_NBUF):
                t = i * _NBUF + j
                slot = j
                nslot = (j + _AHEAD) % _NBUF
                tp = t - (_NBUF - _AHEAD)
                cidp = tp * _NW + w

                @pl.when((tp >= 0) & (cidp <= nchunks - 1))
                def _():
                    wait_scatter(nslot)

                fire_inputs(t + _AHEAD, nslot)
                wait_inputs(t, slot)
                cid = t * _NW + w

                @pl.when(cid <= nchunks - 1)
                def _():
                    compute(slot)
                    fire_scatter(slot)
            return carry

        lax.fori_loop(0, steps // _NBUF, pipe_body, 0)

        for t in range(steps - (_NBUF - _AHEAD), steps):
            cid = t * _NW + w

            @pl.when((t >= 0) & (cid <= nchunks - 1))
            def _():
                wait_scatter(t % _NBUF)

        plsc.subcore_barrier()
        zsl = pl.ds(s * _NSL, _NSL)
        pltpu.sync_copy(y_sh.at[zsl], out_hbm.at[c, zsl])

    return k


def _combine(partials):
    def body(p_ref, o_ref):
        o_ref[...] = p_ref[0] + p_ref[1]

    return pl.pallas_call(
        body,
        out_shape=jax.ShapeDtypeStruct((512, 128), jnp.float32),
    )(partials.reshape(_NC, 512, 128))


def kernel(rows, cols, vals, x):
    nnz = rows.shape[0]
    full = nnz // _C
    tail = nnz - full * _C
    pad = _C - tail if tail else 0
    rows_t = jnp.pad(rows[full * _C:], (0, pad))
    cols_t = jnp.pad(cols[full * _C:], (0, pad))
    vals_t = jnp.pad(vals[full * _C:], (0, pad))
    cols_f = lax.bitcast_convert_type(cols, jnp.float32)
    cols_tf = lax.bitcast_convert_type(cols_t, jnp.float32)
    partials = _spmv(nnz)(rows, cols_f, vals, rows_t, cols_tf, vals_t, x)
    y = _combine(partials).reshape(_N)
    return y.astype(jnp.float64)


# speedup vs baseline: 1.1732x; 1.1732x over previous
import functools

import jax
import jax.numpy as jnp
from jax import lax
from jax.experimental import pallas as pl
from jax.experimental.pallas import tpu as pltpu
from jax.experimental.pallas import tpu_sc as plsc

_N = 65536
_NC = 2
_NS = 16
_NW = _NC * _NS
_C = 4096
_G = _C // 16
_NSL = _N // _NS
_NBUF = 3


@functools.lru_cache(maxsize=None)
def _spmv(nnz):
    full = nnz // _C
    tail = nnz - full * _C
    nchunks = full + (1 if tail else 0)
    steps = -(-nchunks // _NW)
    steps = -(-steps // _NBUF) * _NBUF

    mesh = plsc.VectorSubcoreMesh(core_axis_name="c", subcore_axis_name="s")

    @functools.partial(
        pl.kernel,
        out_type=jax.ShapeDtypeStruct((_NC, _N), jnp.float32),
        mesh=mesh,
        compiler_params=pltpu.CompilerParams(needs_layout_passes=False),
        scratch_types=[
            pltpu.VMEM((_N,), jnp.float32),
            *[pltpu.VMEM((_C,), jnp.int32) for _ in range(_NBUF)],
            *[pltpu.VMEM((_C,), jnp.int32) for _ in range(_NBUF)],
            *[pltpu.VMEM((_C,), jnp.float32) for _ in range(_NBUF)],
            *[pltpu.VMEM((_C,), jnp.float32) for _ in range(_NBUF)],
            pltpu.VMEM_SHARED((_N,), jnp.float32),
            *[pltpu.SemaphoreType.DMA for _ in range(2 * _NBUF)],
        ],
    )
    def k(rows_hbm, cols_hbm, vals_hbm, rows_t, cols_t, vals_t,
          x_hbm, out_hbm,
          x_l, rb0, rb1, rb2, cb0, cb1, cb2, vb0, vb1, vb2, pb0, pb1, pb2,
          y_sh, isem0, isem1, isem2, ssem0, ssem1, ssem2):
        rows_b = (rb0, rb1, rb2)
        cols_b = (cb0, cb1, cb2)
        vals_b = (vb0, vb1, vb2)
        prod_b = (pb0, pb1, pb2)
        in_sems = (isem0, isem1, isem2)
        sc_sems = (ssem0, ssem1, ssem2)
        c = lax.axis_index("c")
        s = lax.axis_index("s")
        w = c * _NS + s

        pltpu.sync_copy(x_hbm, x_l)
        zsl = pl.ds(s * _NSL, _NSL)
        z16 = jnp.zeros((16,), jnp.float32)

        @plsc.parallel_loop(0, _G, 1, unroll=8)
        def _z(g):
            pb0[pl.ds(g * 16, 16)] = z16

        pltpu.sync_copy(pb0, y_sh.at[zsl])
        plsc.subcore_barrier()

        def fire_inputs(t, slot):
            cid = t * _NW + w
            base = cid * _C

            @pl.when(cid < full)
            def _():
                pltpu.async_copy(rows_hbm.at[pl.ds(base, _C)],
                                 rows_b[slot], in_sems[slot])
                pltpu.async_copy(cols_hbm.at[pl.ds(base, _C)],
                                 cols_b[slot], in_sems[slot])
                pltpu.async_copy(vals_hbm.at[pl.ds(base, _C)],
                                 vals_b[slot], in_sems[slot])

            if tail:
                @pl.when(cid == full)
                def _():
                    pltpu.async_copy(rows_t, rows_b[slot], in_sems[slot])
                    pltpu.async_copy(cols_t, cols_b[slot], in_sems[slot])
                    pltpu.async_copy(vals_t, vals_b[slot], in_sems[slot])

        def wait_inputs(t, slot):
            cid = t * _NW + w

            @pl.when(cid <= nchunks - 1)
            def _():
                pltpu.make_async_copy(rows_hbm.at[pl.ds(0, _C)],
                                      rows_b[slot], in_sems[slot]).wait()
                pltpu.make_async_copy(cols_hbm.at[pl.ds(0, _C)],
                                      cols_b[slot], in_sems[slot]).wait()
                pltpu.make_async_copy(vals_hbm.at[pl.ds(0, _C)],
                                      vals_b[slot], in_sems[slot]).wait()

        def compute(slot):
            @plsc.parallel_loop(0, _G, 1, unroll=8)
            def g_body(g):
                sl = pl.ds(g * 16, 16)
                idx = cols_b[slot][sl]
                xv = plsc.load_gather(x_l, [idx])
                prod_b[slot][sl] = xv * vals_b[slot][sl]

        def fire_scatter(slot):
            pltpu.async_copy(prod_b[slot], y_sh.at[rows_b[slot]],
                             sc_sems[slot], add=True)

        def wait_scatter(slot):
            pltpu.make_async_copy(prod_b[slot], y_sh.at[rows_b[slot]],
                                  sc_sems[slot]).wait()

        fire_inputs(0, 0)

        def pipe_body(i, carry):
            for j in range(_NBUF):
                t = i * _NBUF + j
                slot = j
                nslot = (j + 1) % _NBUF
                tp = t - 2
                cidp = tp * _NW + w

                @pl.when((tp >= 0) & (cidp <= nchunks - 1))
                def _():
                    wait_scatter(nslot)

                fire_inputs(t + 1, nslot)
                wait_inputs(t, slot)
                cid = t * _NW + w

                @pl.when(cid <= nchunks - 1)
                def _():
                    compute(slot)
                    fire_scatter(slot)
            return carry

        lax.fori_loop(0, steps // _NBUF, pipe_body, 0)

        for t in (steps - 2, steps - 1):
            cid = t * _NW + w

            @pl.when(cid <= nchunks - 1)
            def _():
                wait_scatter(t % _NBUF)

        plsc.subcore_barrier()
        pltpu.sync_copy(y_sh.at[zsl], out_hbm.at[c, zsl])

    return k


def _combine(partials):
    def body(p_ref, o_ref):
        o_ref[...] = p_ref[0] + p_ref[1]

    return pl.pallas_call(
        body,
        out_shape=jax.ShapeDtypeStruct((_N,), jnp.float32),
    )(partials)


def kernel(rows, cols, vals, x):
    nnz = rows.shape[0]
    full = nnz // _C
    tail = nnz - full * _C
    pad = _C - tail if tail else 0
    rows_t = jnp.pad(rows[full * _C:], (0, pad))
    cols_t = jnp.pad(cols[full * _C:], (0, pad))
    vals_t = jnp.pad(vals[full * _C:], (0, pad))
    partials = _spmv(nnz)(rows, cols, vals, rows_t, cols_t, vals_t, x)
    y = _combine(partials)
    return y.astype(jnp.float64)


# speedup vs baseline: 1.2664x; 1.0795x over previous
import functools

import jax
import jax.numpy as jnp
from jax import lax
from jax.experimental import pallas as pl
from jax.experimental.pallas import tpu as pltpu
from jax.experimental.pallas import tpu_sc as plsc

_N = 65536
_NC = 2
_NS = 16
_NW = _NC * _NS
_C = 4096
_G = _C // 16
_NSL = _N // _NS
_NBUF = 3


@functools.lru_cache(maxsize=None)
def _spmv(nnz):
    full = nnz // _C
    tail = nnz - full * _C
    nchunks = full + (1 if tail else 0)
    steps = -(-nchunks // _NW)
    steps = -(-steps // _NBUF) * _NBUF

    mesh = plsc.VectorSubcoreMesh(core_axis_name="c", subcore_axis_name="s")

    @functools.partial(
        pl.kernel,
        out_type=jax.ShapeDtypeStruct((_NC, _N), jnp.float32),
        mesh=mesh,
        compiler_params=pltpu.CompilerParams(needs_layout_passes=False),
        scratch_types=[
            pltpu.VMEM((_N,), jnp.float32),
            *[pltpu.VMEM((_C,), jnp.int32) for _ in range(_NBUF)],
            *[pltpu.VMEM((_C,), jnp.int32) for _ in range(_NBUF)],
            *[pltpu.VMEM((_C,), jnp.float32) for _ in range(_NBUF)],
            *[pltpu.VMEM((_C,), jnp.float32) for _ in range(_NBUF)],
            pltpu.VMEM_SHARED((_N,), jnp.float32),
            pltpu.VMEM_SHARED((_N,), jnp.float32),
            *[pltpu.SemaphoreType.DMA for _ in range(2 * _NBUF)],
        ],
    )
    def k(rows_hbm, cols_hbm, vals_hbm, rows_t, cols_t, vals_t,
          x_hbm, out_hbm,
          x_l, rb0, rb1, rb2, cb0, cb1, cb2, vb0, vb1, vb2, pb0, pb1, pb2,
          y_sh, x_sh, isem0, isem1, isem2, ssem0, ssem1, ssem2):
        rows_b = (rb0, rb1, rb2)
        cols_b = (cb0, cb1, cb2)
        vals_b = (vb0, vb1, vb2)
        prod_b = (pb0, pb1, pb2)
        in_sems = (isem0, isem1, isem2)
        sc_sems = (ssem0, ssem1, ssem2)
        c = lax.axis_index("c")
        s = lax.axis_index("s")
        w = c * _NS + s

        zsl = pl.ds(s * _NSL, _NSL)
        pltpu.sync_copy(x_hbm.at[zsl], x_sh.at[zsl])
        z16 = jnp.zeros((16,), jnp.float32)

        @plsc.parallel_loop(0, _G, 1, unroll=8)
        def _z(g):
            pb0[pl.ds(g * 16, 16)] = z16

        pltpu.sync_copy(pb0, y_sh.at[zsl])
        plsc.subcore_barrier()
        pltpu.sync_copy(x_sh, x_l)

        def fire_inputs(t, slot):
            cid = t * _NW + w
            base = cid * _C

            @pl.when(cid < full)
            def _():
                pltpu.async_copy(rows_hbm.at[pl.ds(base, _C)],
                                 rows_b[slot], in_sems[slot])
                pltpu.async_copy(cols_hbm.at[pl.ds(base, _C)],
                                 cols_b[slot], in_sems[slot])
                pltpu.async_copy(vals_hbm.at[pl.ds(base, _C)],
                                 vals_b[slot], in_sems[slot])

            if tail:
                @pl.when(cid == full)
                def _():
                    pltpu.async_copy(rows_t, rows_b[slot], in_sems[slot])
                    pltpu.async_copy(cols_t, cols_b[slot], in_sems[slot])
                    pltpu.async_copy(vals_t, vals_b[slot], in_sems[slot])

        def wait_inputs(t, slot):
            cid = t * _NW + w

            @pl.when(cid <= nchunks - 1)
            def _():
                pltpu.make_async_copy(rows_hbm.at[pl.ds(0, _C)],
                                      rows_b[slot], in_sems[slot]).wait()
                pltpu.make_async_copy(cols_hbm.at[pl.ds(0, _C)],
                                      cols_b[slot], in_sems[slot]).wait()
                pltpu.make_async_copy(vals_hbm.at[pl.ds(0, _C)],
                                      vals_b[slot], in_sems[slot]).wait()

        def compute(slot):
            @plsc.parallel_loop(0, _G, 1, unroll=8)
            def g_body(g):
                sl = pl.ds(g * 16, 16)
                idx = cols_b[slot][sl]
                xv = plsc.load_gather(x_l, [idx])
                prod_b[slot][sl] = xv * vals_b[slot][sl]

        def fire_scatter(slot):
            pltpu.async_copy(prod_b[slot], y_sh.at[rows_b[slot]],
                             sc_sems[slot], add=True)

        def wait_scatter(slot):
            pltpu.make_async_copy(prod_b[slot], y_sh.at[rows_b[slot]],
                                  sc_sems[slot]).wait()

        fire_inputs(0, 0)

        def pipe_body(i, carry):
            for j in range(_NBUF):
                t = i * _NBUF + j
                slot = j
                nslot = (j + 1) % _NBUF
                tp = t - 2
                cidp = tp * _NW + w

                @pl.when((tp >= 0) & (cidp <= nchunks - 1))
                def _():
                    wait_scatter(nslot)

                fire_inputs(t + 1, nslot)
                wait_inputs(t, slot)
                cid = t * _NW + w

                @pl.when(cid <= nchunks - 1)
                def _():
                    compute(slot)
                    fire_scatter(slot)
            return carry

        lax.fori_loop(0, steps // _NBUF, pipe_body, 0)

        for t in (steps - 2, steps - 1):
            cid = t * _NW + w

            @pl.when(cid <= nchunks - 1)
            def _():
                wait_scatter(t % _NBUF)

        plsc.subcore_barrier()
        pltpu.sync_copy(y_sh.at[zsl], out_hbm.at[c, zsl])

    return k


def _combine(partials):
    def body(p_ref, o_ref):
        o_ref[...] = p_ref[0] + p_ref[1]

    return pl.pallas_call(
        body,
        out_shape=jax.ShapeDtypeStruct((_N,), jnp.float32),
    )(partials)


def kernel(rows, cols, vals, x):
    nnz = rows.shape[0]
    full = nnz // _C
    tail = nnz - full * _C
    pad = _C - tail if tail else 0
    rows_t = jnp.pad(rows[full * _C:], (0, pad))
    cols_t = jnp.pad(cols[full * _C:], (0, pad))
    vals_t = jnp.pad(vals[full * _C:], (0, pad))
    partials = _spmv(nnz)(rows, cols, vals, rows_t, cols_t, vals_t, x)
    y = _combine(partials)
    return y.astype(jnp.float64)
